# baseline (speedup 1.0000x reference)
import functools

import jax
import jax.numpy as jnp
from jax import lax
from jax.experimental import pallas as pl
from jax.experimental.pallas import tpu as pltpu
from jax.experimental.pallas import tpu_sc as plsc

N, E, D = 10000, 320000, 128
NAF, NBF = 9, 3
AV, BV = 64, 8
H2 = 2 * D
ETAB = 512

NC, NS = 2, 16
NW = NC * NS
HCH = 128
NHC = 80
ECH = 64
NEC = 160
CG = 8
EPW = NHC * HCH
EPAD = EPW * NW
RPT = 632
NPAD = RPT * NS

BA = 2000


def _edge_body(h_hbm, etab_hbm, src_hbm, bid_hbm, dst_hbm, agg_hbm,
               src_v, bid_v, dst_v, hrows, erows, agg_sh, etab_sh,
               sem_h0, sem_h1, sem_e0, sem_e1):
    sem_h = (sem_h0, sem_h1)
    sem_e = (sem_e0, sem_e1)
    c = lax.axis_index("c")
    s = lax.axis_index("s")
    w = s * NC + c

    @pl.when(s == 0)
    def _():
        pltpu.sync_copy(etab_hbm, etab_sh)

    def zrow(r, carry):
        for k in range(D // 16):
            hrows[0, r, pl.ds(k * 16, 16)] = jnp.zeros((16,), jnp.float32)
        return carry
    lax.fori_loop(0, HCH, zrow, 0)
    base = s * RPT
    for off in range(0, RPT, HCH):
        sz = min(HCH, RPT - off)
        pltpu.sync_copy(hrows.at[0, pl.ds(0, sz)], agg_sh.at[pl.ds(base + off, sz)])
    plsc.subcore_barrier()

    def group(g, carry):
        pltpu.sync_copy(src_hbm.at[w, pl.ds(g * CG, CG)], src_v)
        pltpu.sync_copy(bid_hbm.at[w, pl.ds(g * 2 * CG, 2 * CG)], bid_v)
        pltpu.sync_copy(dst_hbm.at[w, pl.ds(g * 2 * CG, 2 * CG)], dst_v)

        cps = [None, None]
        cps[0] = pltpu.async_copy(h_hbm.at[src_v.at[0]], hrows.at[0], sem_h[0])
        for j in range(CG):
            b = j % 2
            if j + 1 < CG:
                nb = (j + 1) % 2
                cps[nb] = pltpu.async_copy(h_hbm.at[src_v.at[j + 1]],
                                           hrows.at[nb], sem_h[nb])
            cpe = pltpu.async_copy(etab_sh.at[bid_v.at[2 * j]], erows, sem_e0)
            cps[b].wait()
            for half in range(2):
                cpe.wait()

                def row(r, carry2, b=b, half=half):
                    for k in range(D // 16):
                        sl = pl.ds(k * 16, 16)
                        rr = half * ECH + r
                        hrows[b, rr, sl] = jnp.maximum(
                            hrows[b, rr, sl] + erows[r, sl], 0.0)
                    return carry2
                lax.fori_loop(0, ECH, row, 0)
                if half == 0:
                    cpe = pltpu.async_copy(etab_sh.at[bid_v.at[2 * j + 1]],
                                           erows, sem_e0)
                pltpu.sync_copy(hrows.at[b, pl.ds(half * ECH, ECH)],
                                agg_sh.at[dst_v.at[2 * j + half]], add=True)
        return carry
    lax.fori_loop(0, NHC // CG, group, 0)

    plsc.subcore_barrier()
    pltpu.sync_copy(agg_sh.at[pl.ds(base, RPT)], agg_hbm.at[c, pl.ds(base, RPT)])


@functools.lru_cache(maxsize=None)
def _edge_call():
    mesh = plsc.VectorSubcoreMesh(
        core_axis_name="c", subcore_axis_name="s", num_cores=NC, num_subcores=NS)
    return pl.kernel(
        _edge_body,
        out_type=jax.ShapeDtypeStruct((NC, NPAD, D), jnp.float32),
        mesh=mesh,
        scratch_types=[
            pltpu.VMEM((CG, HCH), jnp.int32),
            pltpu.VMEM((2 * CG, ECH), jnp.int32),
            pltpu.VMEM((2 * CG, ECH), jnp.int32),
            pltpu.VMEM((2, HCH, D), jnp.float32),
            pltpu.VMEM((ECH, D), jnp.float32),
            pltpu.VMEM_SHARED((NPAD, D), jnp.float32),
            pltpu.VMEM_SHARED((ETAB, D), jnp.float32),
            pltpu.SemaphoreType.DMA,
            pltpu.SemaphoreType.DMA,
            pltpu.SemaphoreType.DMA,
            pltpu.SemaphoreType.DMA,
        ],
    )


def _encode_body(x_ref, tabs_ref, out_ref):
    xv = x_ref[...]
    acc = jnp.zeros((BA, D), jnp.float32)
    for f in range(NAF):
        oh = (xv[:, f][:, None] == lax.broadcasted_iota(jnp.int32, (BA, AV), 1))
        acc += jnp.dot(oh.astype(jnp.float32), tabs_ref[f],
                       preferred_element_type=jnp.float32, precision=lax.Precision.HIGHEST)
    out_ref[...] = acc


def _etab_body(bt_ref, out_ref):
    b_ids = lax.broadcasted_iota(jnp.int32, (ETAB, BV), 0)
    k_ids = lax.broadcasted_iota(jnp.int32, (ETAB, BV), 1)
    ohs = [((b_ids // (BV ** (NBF - 1 - f))) % BV == k_ids).astype(jnp.float32)
           for f in range(NBF)]
    for l in range(3):
        acc = jnp.zeros((ETAB, D), jnp.float32)
        for f in range(NBF):
            acc += jnp.dot(ohs[f], bt_ref[l, f], preferred_element_type=jnp.float32, precision=lax.Precision.HIGHEST)
        out_ref[l] = acc


def _bid_body(ea_ref, out_ref):
    out_ref[...] = ea_ref[0] * (BV * BV) + ea_ref[1] * BV + ea_ref[2]


def _mlp_a_body(h_ref, agg_ref, eps_ref, w1_ref, b1_ref, z1_ref, st_ref):
    z = (1.0 + eps_ref[0, 0]) * h_ref[...] + agg_ref[0] + agg_ref[1]
    z1 = jnp.dot(z, w1_ref[...], preferred_element_type=jnp.float32) + b1_ref[...]
    z1_ref[...] = z1

    @pl.when(pl.program_id(0) == 0)
    def _():
        st_ref[...] = jnp.zeros_like(st_ref)
    s1 = jnp.sum(z1, axis=0, keepdims=True)
    s2 = jnp.sum(z1 * z1, axis=0, keepdims=True)
    st_ref[...] += jnp.concatenate(
        [s1, s2, jnp.zeros((6, z1.shape[1]), jnp.float32)], axis=0)


def _mlp_b_body(z1_ref, st_ref, g_ref, b_ref, w2_ref, b2_ref, z2_ref, st2_ref):
    mean = st_ref[0:1, :] / N
    var = st_ref[1:2, :] / N - mean * mean
    zn = (z1_ref[...] - mean) * lax.rsqrt(var + 1e-5) * g_ref[...] + b_ref[...]
    a = jnp.maximum(zn, 0.0)
    z2 = jnp.dot(a, w2_ref[...], preferred_element_type=jnp.float32) + b2_ref[...]
    z2_ref[...] = z2

    @pl.when(pl.program_id(0) == 0)
    def _():
        st2_ref[...] = jnp.zeros_like(st2_ref)
    s1 = jnp.sum(z2, axis=0, keepdims=True)
    s2 = jnp.sum(z2 * z2, axis=0, keepdims=True)
    st2_ref[...] += jnp.concatenate(
        [s1, s2, jnp.zeros((6, z2.shape[1]), jnp.float32)], axis=0)


def _mlp_c_body(z2_ref, st_ref, g_ref, b_ref, out_ref, *, do_relu):
    mean = st_ref[0:1, :] / N
    var = st_ref[1:2, :] / N - mean * mean
    hn = (z2_ref[...] - mean) * lax.rsqrt(var + 1e-5) * g_ref[...] + b_ref[...]
    out_ref[...] = jnp.maximum(hn, 0.0) if do_relu else hn


def _mlp(h, agg, p, do_relu):
    nb = N // BA
    eps = (p['eps'].astype(jnp.float32)).reshape(1, 1)
    z1, st1 = pl.pallas_call(
        _mlp_a_body,
        grid=(nb,),
        in_specs=[
            pl.BlockSpec((BA, D), lambda i: (i, 0)),
            pl.BlockSpec((NC, BA, D), lambda i: (0, i, 0)),
            pl.BlockSpec((1, 1), lambda i: (0, 0)),
            pl.BlockSpec((D, H2), lambda i: (0, 0)),
            pl.BlockSpec((1, H2), lambda i: (0, 0)),
        ],
        out_specs=[
            pl.BlockSpec((BA, H2), lambda i: (i, 0)),
            pl.BlockSpec((8, H2), lambda i: (0, 0)),
        ],
        out_shape=[
            jax.ShapeDtypeStruct((N, H2), jnp.float32),
            jax.ShapeDtypeStruct((8, H2), jnp.float32),
        ],
    )(h, agg, eps, p['W1'], p['b1'].reshape(1, H2))

    z2, st2 = pl.pallas_call(
        _mlp_b_body,
        grid=(nb,),
        in_specs=[
            pl.BlockSpec((BA, H2), lambda i: (i, 0)),
            pl.BlockSpec((8, H2), lambda i: (0, 0)),
            pl.BlockSpec((1, H2), lambda i: (0, 0)),
            pl.BlockSpec((1, H2), lambda i: (0, 0)),
            pl.BlockSpec((H2, D), lambda i: (0, 0)),
            pl.BlockSpec((1, D), lambda i: (0, 0)),
        ],
        out_specs=[
            pl.BlockSpec((BA, D), lambda i: (i, 0)),
            pl.BlockSpec((8, D), lambda i: (0, 0)),
        ],
        out_shape=[
            jax.ShapeDtypeStruct((N, D), jnp.float32),
            jax.ShapeDtypeStruct((8, D), jnp.float32),
        ],
    )(z1, st1, p['bn1_g'].reshape(1, H2), p['bn1_b'].reshape(1, H2),
      p['W2'], p['b2'].reshape(1, D))

    return pl.pallas_call(
        functools.partial(_mlp_c_body, do_relu=do_relu),
        grid=(nb,),
        in_specs=[
            pl.BlockSpec((BA, D), lambda i: (i, 0)),
            pl.BlockSpec((8, D), lambda i: (0, 0)),
            pl.BlockSpec((1, D), lambda i: (0, 0)),
            pl.BlockSpec((1, D), lambda i: (0, 0)),
        ],
        out_specs=pl.BlockSpec((BA, D), lambda i: (i, 0)),
        out_shape=jax.ShapeDtypeStruct((N, D), jnp.float32),
    )(z2, st2, p['bn_g'].reshape(1, D), p['bn_b'].reshape(1, D))


def kernel(x, edge_index, edge_attr, params):
    x32 = x.astype(jnp.int32)
    src = edge_index[0].astype(jnp.int32)
    dst = edge_index[1].astype(jnp.int32)
    ea = edge_attr.astype(jnp.int32)

    x_pad = jnp.pad(x32, ((0, 0), (0, 16 - NAF)), constant_values=AV)
    atabs = jnp.stack(params['atom_tables'])
    h = pl.pallas_call(
        _encode_body,
        grid=(N // BA,),
        in_specs=[
            pl.BlockSpec((BA, 16), lambda i: (i, 0)),
            pl.BlockSpec((NAF, AV, D), lambda i: (0, 0, 0)),
        ],
        out_specs=pl.BlockSpec((BA, D), lambda i: (i, 0)),
        out_shape=jax.ShapeDtypeStruct((N, D), jnp.float32),
    )(x_pad, atabs)

    bts = jnp.stack([jnp.stack(p['bond_tables']) for p in params['layers']])
    etabs = pl.pallas_call(
        _etab_body,
        in_specs=[pl.BlockSpec((3, NBF, BV, D), lambda: (0, 0, 0, 0))],
        out_specs=pl.BlockSpec((3, ETAB, D), lambda: (0, 0, 0)),
        out_shape=jax.ShapeDtypeStruct((3, ETAB, D), jnp.float32),
    )(bts)

    ea_t = ea.T.reshape(NBF, E // HCH, HCH)
    bid2d = pl.pallas_call(
        _bid_body,
        in_specs=[pl.BlockSpec((NBF, E // HCH, HCH), lambda: (0, 0, 0))],
        out_specs=pl.BlockSpec((E // HCH, HCH), lambda: (0, 0)),
        out_shape=jax.ShapeDtypeStruct((E // HCH, HCH), jnp.int32),
    )(ea_t)

    def chunked(a, fill, nchunk, chsz):
        return jnp.pad(a, (0, EPAD - E), constant_values=fill).reshape(
            NW, nchunk, chsz)

    src_c = chunked(src, 0, NHC, HCH)
    dst_c = chunked(dst, N, NEC, ECH)
    bid_c = chunked(bid2d.reshape(E), 0, NEC, ECH)

    edge_fn = _edge_call()
    for l, p in enumerate(params['layers']):
        agg = edge_fn(h, etabs[l], src_c, bid_c, dst_c)
        h = _mlp(h, agg[:, :N, :], p, do_relu=(l != len(params['layers']) - 1))
    return h


# speedup vs baseline: 1.2438x; 1.2438x over previous
import functools

import jax
import jax.numpy as jnp
from jax import lax
from jax.experimental import pallas as pl
from jax.experimental.pallas import tpu as pltpu
from jax.experimental.pallas import tpu_sc as plsc

N, E, D = 10000, 320000, 128
NAF, NBF = 9, 3
AV, BV = 64, 8
H2 = 2 * D
ETAB = 512

NC, NS = 2, 16
NW = NC * NS
CH = 64
CG = 16
NCHUNK = 160
EPW = NCHUNK * CH
EPAD = EPW * NW
RPT = 632
NPAD = RPT * NS

BA = 2000


def _edge_body(h_hbm, etab_hbm, src_hbm, bid_hbm, dst_hbm, agg_hbm,
               src_v, bid_v, dst_v, hrows, erows, agg_sh, etab_sh,
               sem_h0, sem_h1, sem_e0, sem_e1):
    sem_h = (sem_h0, sem_h1)
    sem_e = (sem_e0, sem_e1)
    c = lax.axis_index("c")
    s = lax.axis_index("s")
    w = s * NC + c

    @pl.when(s == 0)
    def _():
        pltpu.sync_copy(etab_hbm, etab_sh)

    def zrow(r, carry):
        for k in range(D // 16):
            hrows[0, r, pl.ds(k * 16, 16)] = jnp.zeros((16,), jnp.float32)
        return carry
    lax.fori_loop(0, CH, zrow, 0)
    base = s * RPT
    for off in range(0, RPT, CH):
        sz = min(CH, RPT - off)
        pltpu.sync_copy(hrows.at[0, pl.ds(0, sz)], agg_sh.at[pl.ds(base + off, sz)])
    plsc.subcore_barrier()

    def group(g, carry):
        pltpu.sync_copy(src_hbm.at[w, pl.ds(g * CG, CG)], src_v)
        pltpu.sync_copy(bid_hbm.at[w, pl.ds(g * CG, CG)], bid_v)
        pltpu.sync_copy(dst_hbm.at[w, pl.ds(g * CG, CG)], dst_v)

        cps = [None, None]
        cps[0] = (pltpu.async_copy(h_hbm.at[src_v.at[0]], hrows.at[0], sem_h[0]),
                  pltpu.async_copy(etab_sh.at[bid_v.at[0]], erows.at[0], sem_e[0]))
        for j in range(CG):
            b = j % 2
            if j + 1 < CG:
                nb = (j + 1) % 2
                cps[nb] = (
                    pltpu.async_copy(h_hbm.at[src_v.at[j + 1]], hrows.at[nb],
                                     sem_h[nb]),
                    pltpu.async_copy(etab_sh.at[bid_v.at[j + 1]], erows.at[nb],
                                     sem_e[nb]))
            cph, cpe = cps[b]
            cph.wait()
            cpe.wait()

            def row(r, carry2, b=b):
                for k in range(D // 16):
                    sl = pl.ds(k * 16, 16)
                    hrows[b, r, sl] = jnp.maximum(
                        hrows[b, r, sl] + erows[b, r, sl], 0.0)
                return carry2
            lax.fori_loop(0, CH, row, 0)
            pltpu.sync_copy(hrows.at[b], agg_sh.at[dst_v.at[j]], add=True)
        return carry
    lax.fori_loop(0, NCHUNK // CG, group, 0)

    plsc.subcore_barrier()
    pltpu.sync_copy(agg_sh.at[pl.ds(base, RPT)], agg_hbm.at[c, pl.ds(base, RPT)])


@functools.lru_cache(maxsize=None)
def _edge_call():
    mesh = plsc.VectorSubcoreMesh(
        core_axis_name="c", subcore_axis_name="s", num_cores=NC, num_subcores=NS)
    return pl.kernel(
        _edge_body,
        out_type=jax.ShapeDtypeStruct((NC, NPAD, D), jnp.float32),
        mesh=mesh,
        scratch_types=[
            pltpu.VMEM((CG, CH), jnp.int32),
            pltpu.VMEM((CG, CH), jnp.int32),
            pltpu.VMEM((CG, CH), jnp.int32),
            pltpu.VMEM((2, CH, D), jnp.float32),
            pltpu.VMEM((2, CH, D), jnp.float32),
            pltpu.VMEM_SHARED((NPAD, D), jnp.float32),
            pltpu.VMEM_SHARED((ETAB, D), jnp.float32),
            pltpu.SemaphoreType.DMA,
            pltpu.SemaphoreType.DMA,
            pltpu.SemaphoreType.DMA,
            pltpu.SemaphoreType.DMA,
        ],
    )


def _encode_body(x_ref, tabs_ref, out_ref):
    xv = x_ref[...]
    acc = jnp.zeros((BA, D), jnp.float32)
    for f in range(NAF):
        oh = (xv[:, f][:, None] == lax.broadcasted_iota(jnp.int32, (BA, AV), 1))
        acc += jnp.dot(oh.astype(jnp.float32), tabs_ref[f],
                       preferred_element_type=jnp.float32, precision=lax.Precision.HIGHEST)
    out_ref[...] = acc


def _etab_body(bt_ref, out_ref):
    b_ids = lax.broadcasted_iota(jnp.int32, (ETAB, BV), 0)
    k_ids = lax.broadcasted_iota(jnp.int32, (ETAB, BV), 1)
    ohs = [((b_ids // (BV ** (NBF - 1 - f))) % BV == k_ids).astype(jnp.float32)
           for f in range(NBF)]
    for l in range(3):
        acc = jnp.zeros((ETAB, D), jnp.float32)
        for f in range(NBF):
            acc += jnp.dot(ohs[f], bt_ref[l, f], preferred_element_type=jnp.float32, precision=lax.Precision.HIGHEST)
        out_ref[l] = acc


def _bid_body(ea_ref, out_ref):
    out_ref[...] = ea_ref[0] * (BV * BV) + ea_ref[1] * BV + ea_ref[2]


def _mlp_a_body(h_ref, agg_ref, eps_ref, w1_ref, b1_ref, z1_ref, st_ref):
    z = (1.0 + eps_ref[0, 0]) * h_ref[...] + agg_ref[0] + agg_ref[1]
    z1 = jnp.dot(z, w1_ref[...], preferred_element_type=jnp.float32) + b1_ref[...]
    z1_ref[...] = z1

    @pl.when(pl.program_id(0) == 0)
    def _():
        st_ref[...] = jnp.zeros_like(st_ref)
    s1 = jnp.sum(z1, axis=0, keepdims=True)
    s2 = jnp.sum(z1 * z1, axis=0, keepdims=True)
    st_ref[...] += jnp.concatenate(
        [s1, s2, jnp.zeros((6, z1.shape[1]), jnp.float32)], axis=0)


def _mlp_b_body(z1_ref, st_ref, g_ref, b_ref, w2_ref, b2_ref, z2_ref, st2_ref):
    mean = st_ref[0:1, :] / N
    var = st_ref[1:2, :] / N - mean * mean
    zn = (z1_ref[...] - mean) * lax.rsqrt(var + 1e-5) * g_ref[...] + b_ref[...]
    a = jnp.maximum(zn, 0.0)
    z2 = jnp.dot(a, w2_ref[...], preferred_element_type=jnp.float32) + b2_ref[...]
    z2_ref[...] = z2

    @pl.when(pl.program_id(0) == 0)
    def _():
        st2_ref[...] = jnp.zeros_like(st2_ref)
    s1 = jnp.sum(z2, axis=0, keepdims=True)
    s2 = jnp.sum(z2 * z2, axis=0, keepdims=True)
    st2_ref[...] += jnp.concatenate(
        [s1, s2, jnp.zeros((6, z2.shape[1]), jnp.float32)], axis=0)


def _mlp_c_body(z2_ref, st_ref, g_ref, b_ref, out_ref, *, do_relu):
    mean = st_ref[0:1, :] / N
    var = st_ref[1:2, :] / N - mean * mean
    hn = (z2_ref[...] - mean) * lax.rsqrt(var + 1e-5) * g_ref[...] + b_ref[...]
    out_ref[...] = jnp.maximum(hn, 0.0) if do_relu else hn


def _mlp(h, agg, p, do_relu):
    nb = N // BA
    eps = (p['eps'].astype(jnp.float32)).reshape(1, 1)
    z1, st1 = pl.pallas_call(
        _mlp_a_body,
        grid=(nb,),
        in_specs=[
            pl.BlockSpec((BA, D), lambda i: (i, 0)),
            pl.BlockSpec((NC, BA, D), lambda i: (0, i, 0)),
            pl.BlockSpec((1, 1), lambda i: (0, 0)),
            pl.BlockSpec((D, H2), lambda i: (0, 0)),
            pl.BlockSpec((1, H2), lambda i: (0, 0)),
        ],
        out_specs=[
            pl.BlockSpec((BA, H2), lambda i: (i, 0)),
            pl.BlockSpec((8, H2), lambda i: (0, 0)),
        ],
        out_shape=[
            jax.ShapeDtypeStruct((N, H2), jnp.float32),
            jax.ShapeDtypeStruct((8, H2), jnp.float32),
        ],
    )(h, agg, eps, p['W1'], p['b1'].reshape(1, H2))

    z2, st2 = pl.pallas_call(
        _mlp_b_body,
        grid=(nb,),
        in_specs=[
            pl.BlockSpec((BA, H2), lambda i: (i, 0)),
            pl.BlockSpec((8, H2), lambda i: (0, 0)),
            pl.BlockSpec((1, H2), lambda i: (0, 0)),
            pl.BlockSpec((1, H2), lambda i: (0, 0)),
            pl.BlockSpec((H2, D), lambda i: (0, 0)),
            pl.BlockSpec((1, D), lambda i: (0, 0)),
        ],
        out_specs=[
            pl.BlockSpec((BA, D), lambda i: (i, 0)),
            pl.BlockSpec((8, D), lambda i: (0, 0)),
        ],
        out_shape=[
            jax.ShapeDtypeStruct((N, D), jnp.float32),
            jax.ShapeDtypeStruct((8, D), jnp.float32),
        ],
    )(z1, st1, p['bn1_g'].reshape(1, H2), p['bn1_b'].reshape(1, H2),
      p['W2'], p['b2'].reshape(1, D))

    return pl.pallas_call(
        functools.partial(_mlp_c_body, do_relu=do_relu),
        grid=(nb,),
        in_specs=[
            pl.BlockSpec((BA, D), lambda i: (i, 0)),
            pl.BlockSpec((8, D), lambda i: (0, 0)),
            pl.BlockSpec((1, D), lambda i: (0, 0)),
            pl.BlockSpec((1, D), lambda i: (0, 0)),
        ],
        out_specs=pl.BlockSpec((BA, D), lambda i: (i, 0)),
        out_shape=jax.ShapeDtypeStruct((N, D), jnp.float32),
    )(z2, st2, p['bn_g'].reshape(1, D), p['bn_b'].reshape(1, D))


def kernel(x, edge_index, edge_attr, params):
    x32 = x.astype(jnp.int32)
    src = edge_index[0].astype(jnp.int32)
    dst = edge_index[1].astype(jnp.int32)
    ea = edge_attr.astype(jnp.int32)

    x_pad = jnp.pad(x32, ((0, 0), (0, 16 - NAF)), constant_values=AV)
    atabs = jnp.stack(params['atom_tables'])
    h = pl.pallas_call(
        _encode_body,
        grid=(N // BA,),
        in_specs=[
            pl.BlockSpec((BA, 16), lambda i: (i, 0)),
            pl.BlockSpec((NAF, AV, D), lambda i: (0, 0, 0)),
        ],
        out_specs=pl.BlockSpec((BA, D), lambda i: (i, 0)),
        out_shape=jax.ShapeDtypeStruct((N, D), jnp.float32),
    )(x_pad, atabs)

    bts = jnp.stack([jnp.stack(p['bond_tables']) for p in params['layers']])
    etabs = pl.pallas_call(
        _etab_body,
        in_specs=[pl.BlockSpec((3, NBF, BV, D), lambda: (0, 0, 0, 0))],
        out_specs=pl.BlockSpec((3, ETAB, D), lambda: (0, 0, 0)),
        out_shape=jax.ShapeDtypeStruct((3, ETAB, D), jnp.float32),
    )(bts)

    ea_t = ea.T.reshape(NBF, E // CH, CH)
    bid2d = pl.pallas_call(
        _bid_body,
        in_specs=[pl.BlockSpec((NBF, E // CH, CH), lambda: (0, 0, 0))],
        out_specs=pl.BlockSpec((E // CH, CH), lambda: (0, 0)),
        out_shape=jax.ShapeDtypeStruct((E // CH, CH), jnp.int32),
    )(ea_t)

    def chunked(a, fill):
        return jnp.pad(a, (0, EPAD - E), constant_values=fill).reshape(
            NW, NCHUNK, CH)

    src_c = chunked(src, 0)
    dst_c = chunked(dst, N)
    bid_c = chunked(bid2d.reshape(E), 0)

    edge_fn = _edge_call()
    for l, p in enumerate(params['layers']):
        agg = edge_fn(h, etabs[l], src_c, bid_c, dst_c)
        h = _mlp(h, agg[:, :N, :], p, do_relu=(l != len(params['layers']) - 1))
    return h
